# baseline (speedup 1.0000x reference)
import functools

import jax
import jax.numpy as jnp
from jax import lax
from jax.experimental import pallas as pl
from jax.experimental.pallas import tpu as pltpu
from jax.experimental.pallas import tpu_sc as plsc

NC, NS = 2, 16
NW = NC * NS
CHUNK = 128


def _gather_body(cpw, emb_dim, idx_hbm, table_hbm, out_hbm, idx_v, rows_v, sem):
    wid = lax.axis_index("s") * NC + lax.axis_index("c")
    base = wid * cpw
    pltpu.sync_copy(idx_hbm.at[pl.ds(base, cpw)], idx_v)
    copies = [
        pltpu.async_copy(table_hbm.at[idx_v.at[j]], rows_v.at[j], sem)
        for j in range(cpw)
    ]
    for c in copies:
        c.wait()
    pltpu.sync_copy(rows_v, out_hbm.at[pl.ds(base, cpw)])


def _sc_gather(idx, table):
    n_chunks, _ = idx.shape
    emb_dim = table.shape[1]
    cpw = n_chunks // NW
    mesh = plsc.VectorSubcoreMesh(
        core_axis_name="c", subcore_axis_name="s", num_cores=NC, num_subcores=NS
    )
    grab = pl.kernel(
        functools.partial(_gather_body, cpw, emb_dim),
        out_type=jax.ShapeDtypeStruct((n_chunks, CHUNK, emb_dim), jnp.float32),
        mesh=mesh,
        scratch_types=[
            pltpu.VMEM((cpw, CHUNK), jnp.int32),
            pltpu.VMEM((cpw, CHUNK, emb_dim), jnp.float32),
            pltpu.SemaphoreType.DMA,
        ],
        compiler_params=pltpu.CompilerParams(use_tc_tiling_on_sc=False),
    )
    return grab(idx, table)


def _mlp_body(emb_ref, w1_ref, b1_ref, w2_ref, b2_ref, out_ref):
    h = jnp.dot(emb_ref[...], w1_ref[...], preferred_element_type=jnp.float32)
    h = jnp.maximum(h + b1_ref[...], 0.0)
    out_ref[...] = (
        jnp.dot(h, w2_ref[...], preferred_element_type=jnp.float32) + b2_ref[...]
    )


def _tc_mlp(emb, W1, b1, W2, b2, block_b=2048):
    batch, emb_dim = emb.shape
    out_dim = W2.shape[1]
    grid = (batch // block_b,)
    return pl.pallas_call(
        _mlp_body,
        grid=grid,
        in_specs=[
            pl.BlockSpec((block_b, emb_dim), lambda i: (i, 0)),
            pl.BlockSpec((emb_dim, emb_dim), lambda i: (0, 0)),
            pl.BlockSpec((1, emb_dim), lambda i: (0, 0)),
            pl.BlockSpec((emb_dim, out_dim), lambda i: (0, 0)),
            pl.BlockSpec((1, out_dim), lambda i: (0, 0)),
        ],
        out_specs=pl.BlockSpec((block_b, out_dim), lambda i: (i, 0)),
        out_shape=jax.ShapeDtypeStruct((batch, out_dim), jnp.float32),
    )(emb, W1, b1.reshape(1, -1), W2, b2.reshape(1, -1))


def kernel(user_id, table, W1, b1, W2, b2):
    batch = user_id.shape[0]
    idx = user_id.astype(jnp.int32).reshape(batch // CHUNK, CHUNK)
    emb = _sc_gather(idx, table).reshape(batch, table.shape[1])
    return _tc_mlp(emb, W1, b1, W2, b2)


# speedup vs baseline: 1.0014x; 1.0014x over previous
import functools

import jax
import jax.numpy as jnp
from jax import lax
from jax.experimental import pallas as pl
from jax.experimental.pallas import tpu as pltpu
from jax.experimental.pallas import tpu_sc as plsc

NC, NS = 2, 16
NW = NC * NS
CHUNK = 128


def _gather_body(bpw, idx_hbm, table_hbm, out_hbm, idx_v, rows_v, sem):
    wid = lax.axis_index("s") * NC + lax.axis_index("c")
    base = wid * bpw
    pltpu.sync_copy(idx_hbm.at[pl.ds(base, bpw)], idx_v)
    copies = [
        pltpu.async_copy(
            table_hbm.at[idx_v.at[pl.ds(j * CHUNK, CHUNK)]],
            rows_v.at[pl.ds(j * CHUNK, CHUNK)],
            sem,
        )
        for j in range(bpw // CHUNK)
    ]
    for c in copies:
        c.wait()
    pltpu.sync_copy(rows_v, out_hbm.at[pl.ds(base, bpw)])


def _sc_gather(idx, table):
    batch = idx.shape[0]
    emb_dim = table.shape[1]
    bpw = batch // NW
    mesh = plsc.VectorSubcoreMesh(
        core_axis_name="c", subcore_axis_name="s", num_cores=NC, num_subcores=NS
    )
    grab = pl.kernel(
        functools.partial(_gather_body, bpw),
        out_type=jax.ShapeDtypeStruct((batch, emb_dim), jnp.float32),
        mesh=mesh,
        scratch_types=[
            pltpu.VMEM((bpw,), jnp.int32),
            pltpu.VMEM((bpw, emb_dim), jnp.float32),
            pltpu.SemaphoreType.DMA,
        ],
        compiler_params=pltpu.CompilerParams(use_tc_tiling_on_sc=False),
    )
    return grab(idx, table)


def _mlp_body(emb_ref, w1_ref, b1_ref, w2_ref, b2_ref, out_ref):
    h = jnp.dot(emb_ref[...], w1_ref[...], preferred_element_type=jnp.float32)
    h = jnp.maximum(h + b1_ref[...], 0.0)
    out_ref[...] = (
        jnp.dot(h, w2_ref[...], preferred_element_type=jnp.float32) + b2_ref[...]
    )


def _tc_mlp(emb, W1, b1, W2, b2, block_b=2048):
    batch, emb_dim = emb.shape
    out_dim = W2.shape[1]
    grid = (batch // block_b,)
    return pl.pallas_call(
        _mlp_body,
        grid=grid,
        in_specs=[
            pl.BlockSpec((block_b, emb_dim), lambda i: (i, 0)),
            pl.BlockSpec((emb_dim, emb_dim), lambda i: (0, 0)),
            pl.BlockSpec((1, emb_dim), lambda i: (0, 0)),
            pl.BlockSpec((emb_dim, out_dim), lambda i: (0, 0)),
            pl.BlockSpec((1, out_dim), lambda i: (0, 0)),
        ],
        out_specs=pl.BlockSpec((block_b, out_dim), lambda i: (i, 0)),
        out_shape=jax.ShapeDtypeStruct((batch, out_dim), jnp.float32),
    )(emb, W1, b1.reshape(1, -1), W2, b2.reshape(1, -1))


def kernel(user_id, table, W1, b1, W2, b2):
    idx = user_id.astype(jnp.int32)
    emb = _sc_gather(idx, table)
    return _tc_mlp(emb, W1, b1, W2, b2)


# speedup vs baseline: 1.0233x; 1.0219x over previous
import functools

import jax
import jax.numpy as jnp
from jax import lax
from jax.experimental import pallas as pl
from jax.experimental.pallas import tpu as pltpu
from jax.experimental.pallas import tpu_sc as plsc

NC, NS = 2, 16
NW = NC * NS
CHUNK = 128


def _gather_body(
    ppw, half, emb_dim, idx_hbm, table_hbm, out_hbm, idx_v, rows_lo, rows_hi, sem
):
    wid = lax.axis_index("s") * NC + lax.axis_index("c")
    base = wid * ppw
    pltpu.sync_copy(idx_hbm.at[pl.ds(base, ppw)], idx_v.at[pl.ds(0, ppw)])
    pltpu.sync_copy(idx_hbm.at[pl.ds(half + base, ppw)], idx_v.at[pl.ds(ppw, ppw)])
    copies = []
    for c in range(ppw // CHUNK):
        r = pl.ds(c * CHUNK, CHUNK)
        copies.append(
            pltpu.async_copy(
                table_hbm.at[idx_v.at[pl.ds(c * CHUNK, CHUNK)]], rows_lo.at[r], sem
            )
        )
        copies.append(
            pltpu.async_copy(
                table_hbm.at[idx_v.at[pl.ds(ppw + c * CHUNK, CHUNK)]],
                rows_hi.at[r],
                sem,
            )
        )
    for c in copies:
        c.wait()
    rows = pl.ds(base, ppw)
    pltpu.sync_copy(rows_lo, out_hbm.at[rows, pl.ds(0, emb_dim)])
    pltpu.sync_copy(rows_hi, out_hbm.at[rows, pl.ds(emb_dim, emb_dim)])


def _sc_gather_packed(idx, table):
    batch = idx.shape[0]
    emb_dim = table.shape[1]
    half = batch // 2
    ppw = half // NW
    mesh = plsc.VectorSubcoreMesh(
        core_axis_name="c", subcore_axis_name="s", num_cores=NC, num_subcores=NS
    )
    grab = pl.kernel(
        functools.partial(_gather_body, ppw, half, emb_dim),
        out_type=jax.ShapeDtypeStruct((half, 2 * emb_dim), jnp.float32),
        mesh=mesh,
        scratch_types=[
            pltpu.VMEM((2 * ppw,), jnp.int32),
            pltpu.VMEM((ppw, emb_dim), jnp.float32),
            pltpu.VMEM((ppw, emb_dim), jnp.float32),
            pltpu.SemaphoreType.DMA,
        ],
        compiler_params=pltpu.CompilerParams(use_tc_tiling_on_sc=False),
    )
    return grab(idx, table)


def _mlp_body(emb_ref, w1_ref, b1_ref, w2_ref, b2_ref, out_ref):
    x = emb_ref[...]
    d = w1_ref.shape[0]
    for k, xs in enumerate((x[:, :d], x[:, d:])):
        h = jnp.dot(xs, w1_ref[...], preferred_element_type=jnp.float32)
        h = jnp.maximum(h + b1_ref[...], 0.0)
        out_ref[k] = (
            jnp.dot(h, w2_ref[...], preferred_element_type=jnp.float32) + b2_ref[...]
        )


def _tc_mlp_packed(embp, W1, b1, W2, b2, block_b=2048):
    rows, two_d = embp.shape
    emb_dim = two_d // 2
    out_dim = W2.shape[1]
    grid = (rows // block_b,)
    out = pl.pallas_call(
        _mlp_body,
        grid=grid,
        in_specs=[
            pl.BlockSpec((block_b, two_d), lambda i: (i, 0)),
            pl.BlockSpec((emb_dim, emb_dim), lambda i: (0, 0)),
            pl.BlockSpec((1, emb_dim), lambda i: (0, 0)),
            pl.BlockSpec((emb_dim, out_dim), lambda i: (0, 0)),
            pl.BlockSpec((1, out_dim), lambda i: (0, 0)),
        ],
        out_specs=pl.BlockSpec((2, block_b, out_dim), lambda i: (0, i, 0)),
        out_shape=jax.ShapeDtypeStruct((2, rows, out_dim), jnp.float32),
    )(embp, W1, b1.reshape(1, -1), W2, b2.reshape(1, -1))
    return out.reshape(2 * rows, out_dim)


def kernel(user_id, table, W1, b1, W2, b2):
    idx = user_id.astype(jnp.int32)
    embp = _sc_gather_packed(idx, table)
    return _tc_mlp_packed(embp, W1, b1, W2, b2)


# speedup vs baseline: 1.7124x; 1.6735x over previous
import functools

import jax
import jax.numpy as jnp
from jax import lax
from jax.experimental import pallas as pl
from jax.experimental.pallas import tpu as pltpu
from jax.experimental.pallas import tpu_sc as plsc

NC, NS = 2, 16
NW = NC * NS


def _gather_body(bpw, idx_hbm, table_hbm, out_hbm, idx_s, rows_v, sem):
    wid = lax.axis_index("s") * NC + lax.axis_index("c")
    base = wid * bpw
    pltpu.sync_copy(idx_hbm.at[pl.ds(base, bpw)], idx_s)

    @pl.loop(0, bpw // 16)
    def _issue(i):
        vec = idx_s[pl.ds(i * 16, 16)]
        for j in range(16):
            pltpu.async_copy(
                table_hbm.at[pl.ds(vec[j], 1)],
                rows_v.at[pl.ds(i * 16 + j, 1)],
                sem,
            )

    pltpu.make_async_copy(table_hbm.at[pl.ds(0, bpw)], rows_v, sem).wait()
    pltpu.sync_copy(rows_v, out_hbm.at[pl.ds(base, bpw)])


def _sc_gather(idx, table):
    batch = idx.shape[0]
    emb_dim = table.shape[1]
    bpw = batch // NW
    mesh = plsc.VectorSubcoreMesh(
        core_axis_name="c", subcore_axis_name="s", num_cores=NC, num_subcores=NS
    )
    grab = pl.kernel(
        functools.partial(_gather_body, bpw),
        out_type=jax.ShapeDtypeStruct((batch, emb_dim), jnp.float32),
        mesh=mesh,
        scratch_types=[
            pltpu.VMEM((bpw,), jnp.int32),
            pltpu.VMEM((bpw, emb_dim), jnp.float32),
            pltpu.SemaphoreType.DMA,
        ],
    )
    return grab(idx, table)


def _mlp_body(emb_ref, w1_ref, b1_ref, w2_ref, b2_ref, out_ref):
    h = jnp.dot(emb_ref[...], w1_ref[...], preferred_element_type=jnp.float32)
    h = jnp.maximum(h + b1_ref[...], 0.0)
    out_ref[...] = (
        jnp.dot(h, w2_ref[...], preferred_element_type=jnp.float32) + b2_ref[...]
    )


def _tc_mlp(emb, W1, b1, W2, b2, block_b=2048):
    batch, emb_dim = emb.shape
    out_dim = W2.shape[1]
    grid = (batch // block_b,)
    return pl.pallas_call(
        _mlp_body,
        grid=grid,
        in_specs=[
            pl.BlockSpec((block_b, emb_dim), lambda i: (i, 0)),
            pl.BlockSpec((emb_dim, emb_dim), lambda i: (0, 0)),
            pl.BlockSpec((1, emb_dim), lambda i: (0, 0)),
            pl.BlockSpec((emb_dim, out_dim), lambda i: (0, 0)),
            pl.BlockSpec((1, out_dim), lambda i: (0, 0)),
        ],
        out_specs=pl.BlockSpec((block_b, out_dim), lambda i: (i, 0)),
        out_shape=jax.ShapeDtypeStruct((batch, out_dim), jnp.float32),
    )(emb, W1, b1.reshape(1, -1), W2, b2.reshape(1, -1))


def kernel(user_id, table, W1, b1, W2, b2):
    idx = user_id.astype(jnp.int32)
    emb = _sc_gather(idx, table)
    return _tc_mlp(emb, W1, b1, W2, b2)
